# Initial kernel scaffold; baseline (speedup 1.0000x reference)
import jax
import jax.numpy as jnp
from jax.experimental import pallas as pl


def kernel(x, edge_index, W1_rel, W1_root, b1, Wmu_rel, Wmu_root, bmu, Wstd, bstd, Wg1_rel, Wg1_root, bg1, Wg2_rel, Wg2_root, bg2, Wc1, bc1, Wc2, bc2, log_std_p):
    raise NotImplementedError("write your pallas kernel here")


# speedup vs baseline: 6.7681x; 6.7681x over previous
import functools

import jax
import jax.numpy as jnp
from jax import lax
from jax.experimental import pallas as pl
from jax.experimental.pallas import tpu as pltpu
from jax.experimental.pallas import tpu_sc as plsc

N = 10000
NP = 10240
E = 320000
NC = 2
NS = 16
NW = NC * NS
EPW = E // NW
CH = 80
NCH = EPW // CH
NPAIR = (NCH - 1) // 2
RPT = NP // NS

f32 = jnp.float32


def _mesh():
    return plsc.VectorSubcoreMesh(core_axis_name="c", subcore_axis_name="s",
                                  num_cores=NC, num_subcores=NS)


_SC_PARAMS = pltpu.CompilerParams(use_tc_tiling_on_sc=False,
                                  needs_layout_passes=False)


def _seg_body(vals_hbm, src_v, dst_v, rows0, rows1, acc, sem0, sem1):
    pltpu.async_copy(vals_hbm.at[src_v.at[0]], rows0, sem0)

    def pair(j, carry):
        i0 = 2 * j
        pltpu.async_copy(vals_hbm.at[src_v.at[i0 + 1]], rows1, sem1)
        pltpu.make_async_copy(vals_hbm.at[src_v.at[i0]], rows0, sem0).wait()
        pltpu.sync_copy(rows0, acc.at[dst_v.at[i0]], add=True)
        pltpu.async_copy(vals_hbm.at[src_v.at[i0 + 2]], rows0, sem0)
        pltpu.make_async_copy(vals_hbm.at[src_v.at[i0 + 1]], rows1, sem1).wait()
        pltpu.sync_copy(rows1, acc.at[dst_v.at[i0 + 1]], add=True)
        return carry

    lax.fori_loop(0, NPAIR, pair, 0)
    pltpu.make_async_copy(vals_hbm.at[src_v.at[NCH - 1]], rows0, sem0).wait()
    pltpu.sync_copy(rows0, acc.at[dst_v.at[NCH - 1]], add=True)


def _make_seg(D):
    @functools.partial(
        pl.kernel,
        out_type=jax.ShapeDtypeStruct((NC, NP, D), f32),
        mesh=_mesh(),
        compiler_params=_SC_PARAMS,
        scratch_types=[
            pltpu.VMEM((NCH, CH), jnp.int32),
            pltpu.VMEM((NCH, CH), jnp.int32),
            pltpu.VMEM((CH, D), f32),
            pltpu.VMEM((CH, D), f32),
            pltpu.VMEM_SHARED((NP, D), f32),
            pltpu.SemaphoreType.DMA,
            pltpu.SemaphoreType.DMA,
        ],
    )
    def seg(vals_hbm, src_hbm, dst_hbm, zeros_hbm, out_hbm,
            src_v, dst_v, rows0, rows1, acc, sem0, sem1):
        c = lax.axis_index("c")
        s = lax.axis_index("s")
        wid = s * NC + c
        off = pl.multiple_of(s * RPT, 8)
        pltpu.sync_copy(src_hbm.at[wid], src_v)
        pltpu.sync_copy(dst_hbm.at[wid], dst_v)
        pltpu.sync_copy(zeros_hbm.at[pl.ds(off, RPT)],
                        acc.at[pl.ds(off, RPT)])
        plsc.subcore_barrier()
        _seg_body(vals_hbm, src_v, dst_v, rows0, rows1, acc, sem0, sem1)
        plsc.subcore_barrier()
        pltpu.sync_copy(acc.at[pl.ds(off, RPT)],
                        out_hbm.at[c].at[pl.ds(off, RPT)])

    return seg


def _make_gate_cos():
    @functools.partial(
        pl.kernel,
        out_type=(jax.ShapeDtypeStruct((NC, NP, 32), f32),
                  jax.ShapeDtypeStruct((E,), f32)),
        mesh=_mesh(),
        compiler_params=_SC_PARAMS,
        scratch_types=[
            pltpu.VMEM((NCH, CH), jnp.int32),
            pltpu.VMEM((NCH, CH), jnp.int32),
            pltpu.VMEM((CH, 32), f32),
            pltpu.VMEM((CH, 32), f32),
            pltpu.VMEM((CH, 32), f32),
            pltpu.VMEM((CH, 32), f32),
            pltpu.VMEM((EPW,), f32),
            pltpu.VMEM_SHARED((NP, 32), f32),
            pltpu.SemaphoreType.DMA,
            pltpu.SemaphoreType.DMA,
            pltpu.SemaphoreType.DMA,
            pltpu.SemaphoreType.DMA,
        ],
    )
    def gate_cos(z_hbm, zn_hbm, src_hbm, dst_hbm, zeros_hbm,
                 out_hbm, wmu_hbm,
                 src_v, dst_v, g0, g1, znr, znc, wv, acc,
                 sem0, sem1, semr, semc):
        c = lax.axis_index("c")
        s = lax.axis_index("s")
        wid = s * NC + c
        off = pl.multiple_of(s * RPT, 8)
        pltpu.sync_copy(src_hbm.at[wid], src_v)
        pltpu.sync_copy(dst_hbm.at[wid], dst_v)
        pltpu.sync_copy(zeros_hbm.at[pl.ds(off, RPT)],
                        acc.at[pl.ds(off, RPT)])
        plsc.subcore_barrier()
        _seg_body(z_hbm, src_v, dst_v, g0, g1, acc, sem0, sem1)
        plsc.subcore_barrier()
        pltpu.sync_copy(acc.at[pl.ds(off, RPT)],
                        out_hbm.at[c].at[pl.ds(off, RPT)])

        iota = lax.iota(jnp.int32, 16)

        def chunk(i, carry):
            pltpu.async_copy(zn_hbm.at[dst_v.at[i]], znr, semr)
            pltpu.async_copy(zn_hbm.at[src_v.at[i]], znc, semc)
            pltpu.make_async_copy(zn_hbm.at[dst_v.at[i]], znr, semr).wait()
            pltpu.make_async_copy(zn_hbm.at[src_v.at[i]], znc, semc).wait()
            for g in range(CH // 16):
                ir = iota + (g * 16)
                acc_v = None
                for d in range(32):
                    ic = jnp.full((16,), d, jnp.int32)
                    a = plsc.load_gather(znr, [ir, ic])
                    b = plsc.load_gather(znc, [ir, ic])
                    t = a * b
                    acc_v = t if acc_v is None else acc_v + t
                wv[pl.ds(i * CH + g * 16, 16)] = acc_v
            return carry

        lax.fori_loop(0, NCH, chunk, 0)
        pltpu.sync_copy(wv, wmu_hbm.at[pl.ds(wid * EPW, EPW)])

    return gate_cos


_seg128 = _make_seg(128)
_seg64 = _make_seg(64)
_seg16 = _make_seg(16)
_gate_cos = _make_gate_cos()


def _leaky(v):
    return jnp.where(v >= 0, v, 0.2 * v)


def _dot(a, b):
    return jnp.dot(a, b, preferred_element_type=f32)


def _tc1(p_ref, x_ref, wr_ref, wt_ref, b_ref, ws_ref, bs_ref,
         h_ref, zstd_ref):
    agg = p_ref[0, :N] + p_ref[1, :N]
    h = _leaky(_dot(agg, wr_ref[...]) + _dot(x_ref[...], wt_ref[...])
               + b_ref[...][None, :])
    h_ref[...] = h
    zstd_ref[...] = jnp.exp(jnp.tanh(_dot(h, ws_ref[...])
                                     + bs_ref[...][None, :]))


def _tc2(p_ref, h_ref, wr_ref, wt_ref, b_ref, z_ref, zn_ref):
    agg = p_ref[0, :N] + p_ref[1, :N]
    z = jnp.tanh(_dot(agg, wr_ref[...]) + _dot(h_ref[...], wt_ref[...])
                 + b_ref[...][None, :])
    z_ref[...] = z
    na = jnp.maximum(jnp.sqrt(jnp.sum(z * z, axis=1, keepdims=True)), 1e-8)
    zn_ref[...] = z / na


def _tc3(p_ref, z_ref, wr_ref, wt_ref, b_ref, x1_ref):
    agg = p_ref[0, :N] + p_ref[1, :N]
    x1_ref[...] = _leaky(_dot(agg, wr_ref[...]) + _dot(z_ref[...], wt_ref[...])
                         + b_ref[...][None, :])


def _tc4(p_ref, x1_ref, wr_ref, wt_ref, b_ref, z_ref, wc1_ref, bc1_ref,
         wc2_ref, bc2_ref, lsp_ref, y_ref, wstd_ref):
    agg = p_ref[0, :N] + p_ref[1, :N]
    x1 = x1_ref[...]
    gate = (_dot(agg, wr_ref[...]) + _dot(x1, wt_ref[...])
            + b_ref[...][None, :])
    m = jnp.max(gate)
    eg = jnp.exp(gate - m)
    ssum = jnp.sum(eg)
    z = z_ref[...]
    pooled = (jnp.sum(eg * z, axis=0) / ssum).reshape(1, 32)
    y1 = _leaky(_dot(pooled, wc1_ref[...]) + bc1_ref[...][None, :])
    y2 = _dot(y1, wc2_ref[...]) + bc2_ref[...][None, :]
    y2 = y2 - jnp.max(y2, axis=1, keepdims=True)
    ey = jnp.exp(y2)
    y_ref[...] = ey / jnp.sum(ey, axis=1, keepdims=True)
    wstd_ref[...] = jnp.exp(lsp_ref[...])


def _sds(shape):
    return jax.ShapeDtypeStruct(shape, f32)


def kernel(x, edge_index, W1_rel, W1_root, b1, Wmu_rel, Wmu_root, bmu,
           Wstd, bstd, Wg1_rel, Wg1_root, bg1, Wg2_rel, Wg2_root, bg2,
           Wc1, bc1, Wc2, bc2, log_std_p):
    ei = edge_index.astype(jnp.int32)
    src = ei[1].reshape(NW, NCH, CH)
    dst = ei[0].reshape(NW, NCH, CH)
    z128 = jnp.zeros((NP, 128), f32)
    z64 = jnp.zeros((NP, 64), f32)
    z32 = jnp.zeros((NP, 32), f32)
    z16 = jnp.zeros((NP, 16), f32)

    p1 = _seg128(x, src, dst, z128)
    h, z_std = pl.pallas_call(
        _tc1, out_shape=(_sds((N, 64)), _sds((N, 32))))(
        p1, x, W1_rel, W1_root, b1, Wstd, bstd)
    p2 = _seg64(h, src, dst, z64)
    z, zn = pl.pallas_call(
        _tc2, out_shape=(_sds((N, 32)), _sds((N, 32))))(
        p2, h, Wmu_rel, Wmu_root, bmu)
    p3, w_mu = _gate_cos(z, zn, src, dst, z32)
    x1 = pl.pallas_call(
        _tc3, out_shape=_sds((N, 16)))(p3, z, Wg1_rel, Wg1_root, bg1)
    p4 = _seg16(x1, src, dst, z16)
    y, w_std = pl.pallas_call(
        _tc4, out_shape=(_sds((1, 2)), _sds((1,))))(
        p4, x1, Wg2_rel, Wg2_root, bg2, z, Wc1, bc1, Wc2, bc2, log_std_p)
    return (y, w_mu, w_std, z, z, z_std)


# speedup vs baseline: 11.1928x; 1.6538x over previous
import functools

import jax
import jax.numpy as jnp
from jax import lax
from jax.experimental import pallas as pl
from jax.experimental.pallas import tpu as pltpu
from jax.experimental.pallas import tpu_sc as plsc

N = 10000
NP = 10240
E = 320000
NC = 2
NS = 16
NW = NC * NS
EPW = E // NW
CH = 80
NCH = EPW // CH
NPAIR = (NCH - 1) // 2
RPT = NP // NS

f32 = jnp.float32


def _mesh():
    return plsc.VectorSubcoreMesh(core_axis_name="c", subcore_axis_name="s",
                                  num_cores=NC, num_subcores=NS)


_SC_PARAMS = pltpu.CompilerParams(use_tc_tiling_on_sc=False,
                                  needs_layout_passes=False)


def _seg_body(vals_hbm, src_v, dst_v, rows0, rows1, acc, sem0, sem1):
    pltpu.async_copy(vals_hbm.at[src_v.at[0]], rows0, sem0)

    def pair(j, carry):
        i0 = 2 * j
        pltpu.async_copy(vals_hbm.at[src_v.at[i0 + 1]], rows1, sem1)
        pltpu.make_async_copy(vals_hbm.at[src_v.at[i0]], rows0, sem0).wait()
        pltpu.sync_copy(rows0, acc.at[dst_v.at[i0]], add=True)
        pltpu.async_copy(vals_hbm.at[src_v.at[i0 + 2]], rows0, sem0)
        pltpu.make_async_copy(vals_hbm.at[src_v.at[i0 + 1]], rows1, sem1).wait()
        pltpu.sync_copy(rows1, acc.at[dst_v.at[i0 + 1]], add=True)
        return carry

    lax.fori_loop(0, NPAIR, pair, 0)
    pltpu.make_async_copy(vals_hbm.at[src_v.at[NCH - 1]], rows0, sem0).wait()
    pltpu.sync_copy(rows0, acc.at[dst_v.at[NCH - 1]], add=True)


def _make_seg(D):
    @functools.partial(
        pl.kernel,
        out_type=jax.ShapeDtypeStruct((NC, NP, D), f32),
        mesh=_mesh(),
        compiler_params=_SC_PARAMS,
        scratch_types=[
            pltpu.VMEM((NCH, CH), jnp.int32),
            pltpu.VMEM((NCH, CH), jnp.int32),
            pltpu.VMEM((CH, D), f32),
            pltpu.VMEM((CH, D), f32),
            pltpu.VMEM_SHARED((NP, D), f32),
            pltpu.SemaphoreType.DMA,
            pltpu.SemaphoreType.DMA,
        ],
    )
    def seg(vals_hbm, src_hbm, dst_hbm, zeros_hbm, out_hbm,
            src_v, dst_v, rows0, rows1, acc, sem0, sem1):
        c = lax.axis_index("c")
        s = lax.axis_index("s")
        wid = s * NC + c
        off = pl.multiple_of(s * RPT, 8)
        pltpu.sync_copy(src_hbm.at[wid], src_v)
        pltpu.sync_copy(dst_hbm.at[wid], dst_v)
        pltpu.sync_copy(zeros_hbm.at[pl.ds(off, RPT)],
                        acc.at[pl.ds(off, RPT)])
        plsc.subcore_barrier()
        _seg_body(vals_hbm, src_v, dst_v, rows0, rows1, acc, sem0, sem1)
        plsc.subcore_barrier()
        pltpu.sync_copy(acc.at[pl.ds(off, RPT)],
                        out_hbm.at[c].at[pl.ds(off, RPT)])

    return seg


ZNW = 32


def _make_gate_cos():
    @functools.partial(
        pl.kernel,
        out_type=(jax.ShapeDtypeStruct((NC, NP, 32), f32),
                  jax.ShapeDtypeStruct((E,), f32)),
        mesh=_mesh(),
        compiler_params=_SC_PARAMS,
        scratch_types=[
            pltpu.VMEM((NCH, CH), jnp.int32),
            pltpu.VMEM((NCH, CH), jnp.int32),
            pltpu.VMEM((CH, 32), f32),
            pltpu.VMEM((CH, 32), f32),
            pltpu.VMEM((CH, ZNW), f32),
            pltpu.VMEM((CH, ZNW), f32),
            pltpu.VMEM((CH, ZNW), f32),
            pltpu.VMEM((CH, ZNW), f32),
            pltpu.VMEM((EPW,), f32),
            pltpu.VMEM_SHARED((NP, 32), f32),
            pltpu.SemaphoreType.DMA,
            pltpu.SemaphoreType.DMA,
            pltpu.SemaphoreType.DMA,
            pltpu.SemaphoreType.DMA,
            pltpu.SemaphoreType.DMA,
            pltpu.SemaphoreType.DMA,
        ],
    )
    def gate_cos(z_hbm, zn_hbm, src_hbm, dst_hbm, zeros_hbm,
                 out_hbm, wmu_hbm,
                 src_v, dst_v, g0, g1, znr0, znc0, znr1, znc1, wv, acc,
                 sem0, sem1, semr0, semc0, semr1, semc1):
        c = lax.axis_index("c")
        s = lax.axis_index("s")
        wid = s * NC + c
        off = pl.multiple_of(s * RPT, 8)
        pltpu.sync_copy(src_hbm.at[wid], src_v)
        pltpu.sync_copy(dst_hbm.at[wid], dst_v)
        pltpu.sync_copy(zeros_hbm.at[pl.ds(off, RPT)],
                        acc.at[pl.ds(off, RPT)])
        plsc.subcore_barrier()
        _seg_body(z_hbm, src_v, dst_v, g0, g1, acc, sem0, sem1)
        plsc.subcore_barrier()
        pltpu.sync_copy(acc.at[pl.ds(off, RPT)],
                        out_hbm.at[c].at[pl.ds(off, RPT)])

        iota = lax.iota(jnp.int32, 16)

        def start(i, br, bc, sr, sc_):
            pltpu.async_copy(zn_hbm.at[dst_v.at[i]], br, sr)
            pltpu.async_copy(zn_hbm.at[src_v.at[i]], bc, sc_)

        def wait(i, br, bc, sr, sc_):
            pltpu.make_async_copy(zn_hbm.at[dst_v.at[i]], br, sr).wait()
            pltpu.make_async_copy(zn_hbm.at[src_v.at[i]], bc, sc_).wait()

        def compute(i, br, bc):
            for g in range(CH // 16):
                acc_v = jnp.zeros((16,), f32)
                for l in range(16):
                    e = g * 16 + l
                    a0 = br[e, pl.ds(0, 16)]
                    a1 = br[e, pl.ds(16, 16)]
                    b0 = bc[e, pl.ds(0, 16)]
                    b1 = bc[e, pl.ds(16, 16)]
                    s = jnp.sum(a0 * b0 + a1 * b1)
                    acc_v = jnp.where(iota == l, s, acc_v)
                wv[pl.ds(i * CH + g * 16, 16)] = acc_v

        start(0, znr0, znc0, semr0, semc0)

        def pair(j, carry):
            i0 = 2 * j
            start(i0 + 1, znr1, znc1, semr1, semc1)
            wait(i0, znr0, znc0, semr0, semc0)
            compute(i0, znr0, znc0)
            start(i0 + 2, znr0, znc0, semr0, semc0)
            wait(i0 + 1, znr1, znc1, semr1, semc1)
            compute(i0 + 1, znr1, znc1)
            return carry

        lax.fori_loop(0, NPAIR, pair, 0)
        wait(NCH - 1, znr0, znc0, semr0, semc0)
        compute(NCH - 1, znr0, znc0)
        pltpu.sync_copy(wv, wmu_hbm.at[pl.ds(wid * EPW, EPW)])

    return gate_cos


_seg128 = _make_seg(128)
_seg64 = _make_seg(64)
_seg16 = _make_seg(16)
_gate_cos = _make_gate_cos()


def _leaky(v):
    return jnp.where(v >= 0, v, 0.2 * v)


def _dot(a, b):
    return jnp.dot(a, b, preferred_element_type=f32)


def _tc1(p_ref, x_ref, wr_ref, wt_ref, b_ref, ws_ref, bs_ref,
         h_ref, zstd_ref):
    agg = p_ref[0, :N] + p_ref[1, :N]
    h = _leaky(_dot(agg, wr_ref[...]) + _dot(x_ref[...], wt_ref[...])
               + b_ref[...][None, :])
    h_ref[...] = h
    zstd_ref[...] = jnp.exp(jnp.tanh(_dot(h, ws_ref[...])
                                     + bs_ref[...][None, :]))


def _tc2(p_ref, h_ref, wr_ref, wt_ref, b_ref, z_ref, zn_ref):
    agg = p_ref[0, :N] + p_ref[1, :N]
    z = jnp.tanh(_dot(agg, wr_ref[...]) + _dot(h_ref[...], wt_ref[...])
                 + b_ref[...][None, :])
    z_ref[...] = z
    na = jnp.maximum(jnp.sqrt(jnp.sum(z * z, axis=1, keepdims=True)), 1e-8)
    zn_ref[...] = z / na


def _tc3(p_ref, z_ref, wr_ref, wt_ref, b_ref, x1_ref):
    agg = p_ref[0, :N] + p_ref[1, :N]
    x1_ref[...] = _leaky(_dot(agg, wr_ref[...]) + _dot(z_ref[...], wt_ref[...])
                         + b_ref[...][None, :])


def _tc4(p_ref, x1_ref, wr_ref, wt_ref, b_ref, z_ref, wc1_ref, bc1_ref,
         wc2_ref, bc2_ref, lsp_ref, y_ref, wstd_ref):
    agg = p_ref[0, :N] + p_ref[1, :N]
    x1 = x1_ref[...]
    gate = (_dot(agg, wr_ref[...]) + _dot(x1, wt_ref[...])
            + b_ref[...][None, :])
    m = jnp.max(gate)
    eg = jnp.exp(gate - m)
    ssum = jnp.sum(eg)
    z = z_ref[...]
    pooled = (jnp.sum(eg * z, axis=0) / ssum).reshape(1, 32)
    y1 = _leaky(_dot(pooled, wc1_ref[...]) + bc1_ref[...][None, :])
    y2 = _dot(y1, wc2_ref[...]) + bc2_ref[...][None, :]
    y2 = y2 - jnp.max(y2, axis=1, keepdims=True)
    ey = jnp.exp(y2)
    y_ref[...] = ey / jnp.sum(ey, axis=1, keepdims=True)
    wstd_ref[...] = jnp.exp(lsp_ref[...])


def _sds(shape):
    return jax.ShapeDtypeStruct(shape, f32)


def kernel(x, edge_index, W1_rel, W1_root, b1, Wmu_rel, Wmu_root, bmu,
           Wstd, bstd, Wg1_rel, Wg1_root, bg1, Wg2_rel, Wg2_root, bg2,
           Wc1, bc1, Wc2, bc2, log_std_p):
    ei = edge_index.astype(jnp.int32)
    src = ei[1].reshape(NW, NCH, CH)
    dst = ei[0].reshape(NW, NCH, CH)
    z128 = jnp.zeros((NP, 128), f32)
    z64 = jnp.zeros((NP, 64), f32)
    z32 = jnp.zeros((NP, 32), f32)
    z16 = jnp.zeros((NP, 16), f32)

    p1 = _seg128(x, src, dst, z128)
    h, z_std = pl.pallas_call(
        _tc1, out_shape=(_sds((N, 64)), _sds((N, 32))))(
        p1, x, W1_rel, W1_root, b1, Wstd, bstd)
    p2 = _seg64(h, src, dst, z64)
    z, zn = pl.pallas_call(
        _tc2, out_shape=(_sds((N, 32)), _sds((N, ZNW))))(
        p2, h, Wmu_rel, Wmu_root, bmu)
    p3, w_mu = _gate_cos(z, zn, src, dst, z32)
    x1 = pl.pallas_call(
        _tc3, out_shape=_sds((N, 16)))(p3, z, Wg1_rel, Wg1_root, bg1)
    p4 = _seg16(x1, src, dst, z16)
    y, w_std = pl.pallas_call(
        _tc4, out_shape=(_sds((1, 2)), _sds((1,))))(
        p4, x1, Wg2_rel, Wg2_root, bg2, z, Wc1, bc1, Wc2, bc2, log_std_p)
    return (y, w_mu, w_std, z, z, z_std)


# speedup vs baseline: 12.2409x; 1.0936x over previous
import functools

import jax
import jax.numpy as jnp
from jax import lax
from jax.experimental import pallas as pl
from jax.experimental.pallas import tpu as pltpu
from jax.experimental.pallas import tpu_sc as plsc

N = 10000
NP = 10240
E = 320000
NC = 2
NS = 16
NW = NC * NS
EPW = E // NW
CH = 80
NCH = EPW // CH
NPAIR = (NCH - 1) // 2
RPT = NP // NS

f32 = jnp.float32


def _mesh():
    return plsc.VectorSubcoreMesh(core_axis_name="c", subcore_axis_name="s",
                                  num_cores=NC, num_subcores=NS)


_SC_PARAMS = pltpu.CompilerParams(use_tc_tiling_on_sc=False,
                                  needs_layout_passes=False)


def _seg_body(vals_hbm, src_v, dst_v, rows0, rows1, acc, sem0, sem1):
    pltpu.async_copy(vals_hbm.at[src_v.at[0]], rows0, sem0)

    def pair(j, carry):
        i0 = 2 * j
        pltpu.async_copy(vals_hbm.at[src_v.at[i0 + 1]], rows1, sem1)
        pltpu.make_async_copy(vals_hbm.at[src_v.at[i0]], rows0, sem0).wait()
        pltpu.sync_copy(rows0, acc.at[dst_v.at[i0]], add=True)
        pltpu.async_copy(vals_hbm.at[src_v.at[i0 + 2]], rows0, sem0)
        pltpu.make_async_copy(vals_hbm.at[src_v.at[i0 + 1]], rows1, sem1).wait()
        pltpu.sync_copy(rows1, acc.at[dst_v.at[i0 + 1]], add=True)
        return carry

    lax.fori_loop(0, NPAIR, pair, 0)
    pltpu.make_async_copy(vals_hbm.at[src_v.at[NCH - 1]], rows0, sem0).wait()
    pltpu.sync_copy(rows0, acc.at[dst_v.at[NCH - 1]], add=True)


def _make_seg(D):
    @functools.partial(
        pl.kernel,
        out_type=jax.ShapeDtypeStruct((NC, NP, D), f32),
        mesh=_mesh(),
        compiler_params=_SC_PARAMS,
        scratch_types=[
            pltpu.VMEM((NCH, CH), jnp.int32),
            pltpu.VMEM((NCH, CH), jnp.int32),
            pltpu.VMEM((CH, D), f32),
            pltpu.VMEM((CH, D), f32),
            pltpu.VMEM_SHARED((NP, D), f32),
            pltpu.SemaphoreType.DMA,
            pltpu.SemaphoreType.DMA,
        ],
    )
    def seg(vals_hbm, src_hbm, dst_hbm, zeros_hbm, out_hbm,
            src_v, dst_v, rows0, rows1, acc, sem0, sem1):
        c = lax.axis_index("c")
        s = lax.axis_index("s")
        wid = s * NC + c
        off = pl.multiple_of(s * RPT, 8)
        pltpu.sync_copy(src_hbm.at[wid], src_v)
        pltpu.sync_copy(dst_hbm.at[wid], dst_v)
        pltpu.sync_copy(zeros_hbm.at[pl.ds(off, RPT)],
                        acc.at[pl.ds(off, RPT)])
        plsc.subcore_barrier()
        _seg_body(vals_hbm, src_v, dst_v, rows0, rows1, acc, sem0, sem1)
        plsc.subcore_barrier()
        pltpu.sync_copy(acc.at[pl.ds(off, RPT)],
                        out_hbm.at[c].at[pl.ds(off, RPT)])

    return seg


ZNW = 32


def _make_gate_cos():
    @functools.partial(
        pl.kernel,
        out_type=(jax.ShapeDtypeStruct((NC, NP, 32), f32),
                  jax.ShapeDtypeStruct((E,), f32)),
        mesh=_mesh(),
        compiler_params=_SC_PARAMS,
        scratch_types=[
            pltpu.VMEM((NCH, CH), jnp.int32),
            pltpu.VMEM((NCH, CH), jnp.int32),
            pltpu.VMEM((CH, 32), f32),
            pltpu.VMEM((CH, 32), f32),
            pltpu.VMEM((CH, ZNW), f32),
            pltpu.VMEM((CH, ZNW), f32),
            pltpu.VMEM((CH, ZNW), f32),
            pltpu.VMEM((CH, ZNW), f32),
            pltpu.VMEM((EPW,), f32),
            pltpu.VMEM_SHARED((NP, 32), f32),
            pltpu.SemaphoreType.DMA,
            pltpu.SemaphoreType.DMA,
            pltpu.SemaphoreType.DMA,
            pltpu.SemaphoreType.DMA,
            pltpu.SemaphoreType.DMA,
            pltpu.SemaphoreType.DMA,
        ],
    )
    def gate_cos(z_hbm, zn_hbm, src_hbm, dst_hbm, zeros_hbm,
                 out_hbm, wmu_hbm,
                 src_v, dst_v, g0, g1, znr0, znc0, znr1, znc1, wv, acc,
                 sem0, sem1, semr0, semc0, semr1, semc1):
        c = lax.axis_index("c")
        s = lax.axis_index("s")
        wid = s * NC + c
        off = pl.multiple_of(s * RPT, 8)
        pltpu.sync_copy(src_hbm.at[wid], src_v)
        pltpu.sync_copy(dst_hbm.at[wid], dst_v)
        pltpu.sync_copy(zeros_hbm.at[pl.ds(off, RPT)],
                        acc.at[pl.ds(off, RPT)])
        plsc.subcore_barrier()

        iota = lax.iota(jnp.int32, 16)

        def zg_start(i, b, s):
            pltpu.async_copy(z_hbm.at[src_v.at[i]], b, s)

        def zg_wait(i, b, s):
            pltpu.make_async_copy(z_hbm.at[src_v.at[i]], b, s).wait()

        def cs_start(i, br, bc, sr, sc_):
            pltpu.async_copy(zn_hbm.at[dst_v.at[i]], br, sr)
            pltpu.async_copy(zn_hbm.at[src_v.at[i]], bc, sc_)

        def cs_wait(i, br, bc, sr, sc_):
            pltpu.make_async_copy(zn_hbm.at[dst_v.at[i]], br, sr).wait()
            pltpu.make_async_copy(zn_hbm.at[src_v.at[i]], bc, sc_).wait()

        def cos_compute(i, br, bc):
            for g in range(CH // 16):
                acc_v = jnp.zeros((16,), f32)
                for l in range(16):
                    e = g * 16 + l
                    a0 = br[e, pl.ds(0, 16)]
                    a1 = br[e, pl.ds(16, 16)]
                    b0 = bc[e, pl.ds(0, 16)]
                    b1 = bc[e, pl.ds(16, 16)]
                    s_ = jnp.sum(a0 * b0 + a1 * b1)
                    acc_v = jnp.where(iota == l, s_, acc_v)
                wv[pl.ds(i * CH + g * 16, 16)] = acc_v

        zg_start(0, g0, sem0)
        cs_start(0, znr0, znc0, semr0, semc0)

        def pair(j, carry):
            i0 = 2 * j
            zg_start(i0 + 1, g1, sem1)
            cs_start(i0 + 1, znr1, znc1, semr1, semc1)
            zg_wait(i0, g0, sem0)
            pltpu.sync_copy(g0, acc.at[dst_v.at[i0]], add=True)
            cs_wait(i0, znr0, znc0, semr0, semc0)
            cos_compute(i0, znr0, znc0)
            zg_start(i0 + 2, g0, sem0)
            cs_start(i0 + 2, znr0, znc0, semr0, semc0)
            zg_wait(i0 + 1, g1, sem1)
            pltpu.sync_copy(g1, acc.at[dst_v.at[i0 + 1]], add=True)
            cs_wait(i0 + 1, znr1, znc1, semr1, semc1)
            cos_compute(i0 + 1, znr1, znc1)
            return carry

        lax.fori_loop(0, NPAIR, pair, 0)
        zg_wait(NCH - 1, g0, sem0)
        pltpu.sync_copy(g0, acc.at[dst_v.at[NCH - 1]], add=True)
        cs_wait(NCH - 1, znr0, znc0, semr0, semc0)
        cos_compute(NCH - 1, znr0, znc0)

        plsc.subcore_barrier()
        pltpu.sync_copy(acc.at[pl.ds(off, RPT)],
                        out_hbm.at[c].at[pl.ds(off, RPT)])
        pltpu.sync_copy(wv, wmu_hbm.at[pl.ds(wid * EPW, EPW)])

    return gate_cos


_seg128 = _make_seg(128)
_seg64 = _make_seg(64)
_seg16 = _make_seg(16)
_gate_cos = _make_gate_cos()


def _leaky(v):
    return jnp.where(v >= 0, v, 0.2 * v)


def _dot(a, b):
    return jnp.dot(a, b, preferred_element_type=f32)


def _tc1(p_ref, x_ref, wr_ref, wt_ref, b_ref, ws_ref, bs_ref,
         h_ref, zstd_ref):
    agg = p_ref[0, :N] + p_ref[1, :N]
    h = _leaky(_dot(agg, wr_ref[...]) + _dot(x_ref[...], wt_ref[...])
               + b_ref[...][None, :])
    h_ref[...] = h
    zstd_ref[...] = jnp.exp(jnp.tanh(_dot(h, ws_ref[...])
                                     + bs_ref[...][None, :]))


def _tc2(p_ref, h_ref, wr_ref, wt_ref, b_ref, z_ref, zn_ref):
    agg = p_ref[0, :N] + p_ref[1, :N]
    z = jnp.tanh(_dot(agg, wr_ref[...]) + _dot(h_ref[...], wt_ref[...])
                 + b_ref[...][None, :])
    z_ref[...] = z
    na = jnp.maximum(jnp.sqrt(jnp.sum(z * z, axis=1, keepdims=True)), 1e-8)
    zn_ref[...] = z / na


def _tc3(p_ref, z_ref, wr_ref, wt_ref, b_ref, x1_ref):
    agg = p_ref[0, :N] + p_ref[1, :N]
    x1_ref[...] = _leaky(_dot(agg, wr_ref[...]) + _dot(z_ref[...], wt_ref[...])
                         + b_ref[...][None, :])


def _tc4(p_ref, x1_ref, wr_ref, wt_ref, b_ref, z_ref, wc1_ref, bc1_ref,
         wc2_ref, bc2_ref, lsp_ref, y_ref, wstd_ref):
    agg = p_ref[0, :N] + p_ref[1, :N]
    x1 = x1_ref[...]
    gate = (_dot(agg, wr_ref[...]) + _dot(x1, wt_ref[...])
            + b_ref[...][None, :])
    m = jnp.max(gate)
    eg = jnp.exp(gate - m)
    ssum = jnp.sum(eg)
    z = z_ref[...]
    pooled = (jnp.sum(eg * z, axis=0) / ssum).reshape(1, 32)
    y1 = _leaky(_dot(pooled, wc1_ref[...]) + bc1_ref[...][None, :])
    y2 = _dot(y1, wc2_ref[...]) + bc2_ref[...][None, :]
    y2 = y2 - jnp.max(y2, axis=1, keepdims=True)
    ey = jnp.exp(y2)
    y_ref[...] = ey / jnp.sum(ey, axis=1, keepdims=True)
    wstd_ref[...] = jnp.exp(lsp_ref[...])


def _sds(shape):
    return jax.ShapeDtypeStruct(shape, f32)


def kernel(x, edge_index, W1_rel, W1_root, b1, Wmu_rel, Wmu_root, bmu,
           Wstd, bstd, Wg1_rel, Wg1_root, bg1, Wg2_rel, Wg2_root, bg2,
           Wc1, bc1, Wc2, bc2, log_std_p):
    ei = edge_index.astype(jnp.int32)
    src = ei[1].reshape(NW, NCH, CH)
    dst = ei[0].reshape(NW, NCH, CH)
    z128 = jnp.zeros((NP, 128), f32)
    z64 = jnp.zeros((NP, 64), f32)
    z32 = jnp.zeros((NP, 32), f32)
    z16 = jnp.zeros((NP, 16), f32)

    p1 = _seg128(x, src, dst, z128)
    h, z_std = pl.pallas_call(
        _tc1, out_shape=(_sds((N, 64)), _sds((N, 32))))(
        p1, x, W1_rel, W1_root, b1, Wstd, bstd)
    p2 = _seg64(h, src, dst, z64)
    z, zn = pl.pallas_call(
        _tc2, out_shape=(_sds((N, 32)), _sds((N, ZNW))))(
        p2, h, Wmu_rel, Wmu_root, bmu)
    p3, w_mu = _gate_cos(z, zn, src, dst, z32)
    x1 = pl.pallas_call(
        _tc3, out_shape=_sds((N, 16)))(p3, z, Wg1_rel, Wg1_root, bg1)
    p4 = _seg16(x1, src, dst, z16)
    y, w_std = pl.pallas_call(
        _tc4, out_shape=(_sds((1, 2)), _sds((1,))))(
        p4, x1, Wg2_rel, Wg2_root, bg2, z, Wc1, bc1, Wc2, bc2, log_std_p)
    return (y, w_mu, w_std, z, z, z_std)


# speedup vs baseline: 12.4114x; 1.0139x over previous
import functools

import jax
import jax.numpy as jnp
from jax import lax
from jax.experimental import pallas as pl
from jax.experimental.pallas import tpu as pltpu
from jax.experimental.pallas import tpu_sc as plsc

N = 10000
NP = 10240
E = 320000
NC = 2
NS = 16
NW = NC * NS
EPW = E // NW
CH = 80
NCH = EPW // CH
NPAIR = (NCH - 1) // 2
RPT = NP // NS

f32 = jnp.float32


def _mesh():
    return plsc.VectorSubcoreMesh(core_axis_name="c", subcore_axis_name="s",
                                  num_cores=NC, num_subcores=NS)


_SC_PARAMS = pltpu.CompilerParams(use_tc_tiling_on_sc=False,
                                  needs_layout_passes=False)


def _seg_body(vals_hbm, src_v, dst_v, rows0, rows1, acc, sem0, sem1):
    pltpu.async_copy(vals_hbm.at[src_v.at[0]], rows0, sem0)

    def pair(j, carry):
        i0 = 2 * j
        pltpu.async_copy(vals_hbm.at[src_v.at[i0 + 1]], rows1, sem1)
        pltpu.make_async_copy(vals_hbm.at[src_v.at[i0]], rows0, sem0).wait()
        pltpu.sync_copy(rows0, acc.at[dst_v.at[i0]], add=True)
        pltpu.async_copy(vals_hbm.at[src_v.at[i0 + 2]], rows0, sem0)
        pltpu.make_async_copy(vals_hbm.at[src_v.at[i0 + 1]], rows1, sem1).wait()
        pltpu.sync_copy(rows1, acc.at[dst_v.at[i0 + 1]], add=True)
        return carry

    lax.fori_loop(0, NPAIR, pair, 0)
    pltpu.make_async_copy(vals_hbm.at[src_v.at[NCH - 1]], rows0, sem0).wait()
    pltpu.sync_copy(rows0, acc.at[dst_v.at[NCH - 1]], add=True)


def _make_seg(D):
    @functools.partial(
        pl.kernel,
        out_type=jax.ShapeDtypeStruct((NC, NP, D), f32),
        mesh=_mesh(),
        compiler_params=_SC_PARAMS,
        scratch_types=[
            pltpu.VMEM((NCH, CH), jnp.int32),
            pltpu.VMEM((NCH, CH), jnp.int32),
            pltpu.VMEM((CH, D), f32),
            pltpu.VMEM((CH, D), f32),
            pltpu.VMEM_SHARED((NP, D), f32),
            pltpu.SemaphoreType.DMA,
            pltpu.SemaphoreType.DMA,
            pltpu.SemaphoreType.DMA,
        ],
    )
    def seg(vals_hbm, src_hbm, dst_hbm, zeros_hbm, out_hbm,
            src_v, dst_v, rows0, rows1, acc, sem0, sem1, semz):
        c = lax.axis_index("c")
        s = lax.axis_index("s")
        wid = s * NC + c
        off = pl.multiple_of(s * RPT, 8)
        pltpu.async_copy(src_hbm.at[wid], src_v, sem0)
        pltpu.async_copy(dst_hbm.at[wid], dst_v, sem1)
        pltpu.async_copy(zeros_hbm.at[pl.ds(off, RPT)],
                        acc.at[pl.ds(off, RPT)], semz)
        pltpu.make_async_copy(src_hbm.at[wid], src_v, sem0).wait()
        pltpu.make_async_copy(dst_hbm.at[wid], dst_v, sem1).wait()
        pltpu.make_async_copy(zeros_hbm.at[pl.ds(off, RPT)],
                              acc.at[pl.ds(off, RPT)], semz).wait()
        plsc.subcore_barrier()
        _seg_body(vals_hbm, src_v, dst_v, rows0, rows1, acc, sem0, sem1)
        plsc.subcore_barrier()
        pltpu.sync_copy(acc.at[pl.ds(off, RPT)],
                        out_hbm.at[c].at[pl.ds(off, RPT)])

    return seg


ZNW = 32


def _make_gate_cos():
    @functools.partial(
        pl.kernel,
        out_type=(jax.ShapeDtypeStruct((NC, NP, 32), f32),
                  jax.ShapeDtypeStruct((E,), f32)),
        mesh=_mesh(),
        compiler_params=_SC_PARAMS,
        scratch_types=[
            pltpu.VMEM((NCH, CH), jnp.int32),
            pltpu.VMEM((NCH, CH), jnp.int32),
            pltpu.VMEM((CH, 32), f32),
            pltpu.VMEM((CH, 32), f32),
            pltpu.VMEM((CH, ZNW), f32),
            pltpu.VMEM((CH, ZNW), f32),
            pltpu.VMEM((CH, ZNW), f32),
            pltpu.VMEM((CH, ZNW), f32),
            pltpu.VMEM((EPW,), f32),
            pltpu.VMEM_SHARED((NP, 32), f32),
            pltpu.SemaphoreType.DMA,
            pltpu.SemaphoreType.DMA,
            pltpu.SemaphoreType.DMA,
            pltpu.SemaphoreType.DMA,
            pltpu.SemaphoreType.DMA,
            pltpu.SemaphoreType.DMA,
        ],
    )
    def gate_cos(z_hbm, zn_hbm, src_hbm, dst_hbm, zeros_hbm,
                 out_hbm, wmu_hbm,
                 src_v, dst_v, g0, g1, znr0, znc0, znr1, znc1, wv, acc,
                 sem0, sem1, semr0, semc0, semr1, semc1):
        c = lax.axis_index("c")
        s = lax.axis_index("s")
        wid = s * NC + c
        off = pl.multiple_of(s * RPT, 8)
        pltpu.async_copy(src_hbm.at[wid], src_v, sem0)
        pltpu.async_copy(dst_hbm.at[wid], dst_v, sem1)
        pltpu.async_copy(zeros_hbm.at[pl.ds(off, RPT)],
                        acc.at[pl.ds(off, RPT)], semr0)
        pltpu.make_async_copy(src_hbm.at[wid], src_v, sem0).wait()
        pltpu.make_async_copy(dst_hbm.at[wid], dst_v, sem1).wait()
        pltpu.make_async_copy(zeros_hbm.at[pl.ds(off, RPT)],
                              acc.at[pl.ds(off, RPT)], semr0).wait()
        plsc.subcore_barrier()

        iota = lax.iota(jnp.int32, 16)

        def zg_start(i, b, s):
            pltpu.async_copy(z_hbm.at[src_v.at[i]], b, s)

        def zg_wait(i, b, s):
            pltpu.make_async_copy(z_hbm.at[src_v.at[i]], b, s).wait()

        def cs_start(i, br, bc, sr, sc_):
            pltpu.async_copy(zn_hbm.at[dst_v.at[i]], br, sr)
            pltpu.async_copy(zn_hbm.at[src_v.at[i]], bc, sc_)

        def cs_wait(i, br, bc, sr, sc_):
            pltpu.make_async_copy(zn_hbm.at[dst_v.at[i]], br, sr).wait()
            pltpu.make_async_copy(zn_hbm.at[src_v.at[i]], bc, sc_).wait()

        def cos_compute(i, br, bc):
            for g in range(CH // 16):
                acc_v = jnp.zeros((16,), f32)
                for l in range(16):
                    e = g * 16 + l
                    a0 = br[e, pl.ds(0, 16)]
                    a1 = br[e, pl.ds(16, 16)]
                    b0 = bc[e, pl.ds(0, 16)]
                    b1 = bc[e, pl.ds(16, 16)]
                    s_ = jnp.sum(a0 * b0 + a1 * b1)
                    acc_v = jnp.where(iota == l, s_, acc_v)
                wv[pl.ds(i * CH + g * 16, 16)] = acc_v

        zg_start(0, g0, sem0)
        cs_start(0, znr0, znc0, semr0, semc0)

        def pair(j, carry):
            i0 = 2 * j
            zg_start(i0 + 1, g1, sem1)
            cs_start(i0 + 1, znr1, znc1, semr1, semc1)
            zg_wait(i0, g0, sem0)
            pltpu.sync_copy(g0, acc.at[dst_v.at[i0]], add=True)
            cs_wait(i0, znr0, znc0, semr0, semc0)
            cos_compute(i0, znr0, znc0)
            zg_start(i0 + 2, g0, sem0)
            cs_start(i0 + 2, znr0, znc0, semr0, semc0)
            zg_wait(i0 + 1, g1, sem1)
            pltpu.sync_copy(g1, acc.at[dst_v.at[i0 + 1]], add=True)
            cs_wait(i0 + 1, znr1, znc1, semr1, semc1)
            cos_compute(i0 + 1, znr1, znc1)
            return carry

        lax.fori_loop(0, NPAIR, pair, 0)
        zg_wait(NCH - 1, g0, sem0)
        pltpu.sync_copy(g0, acc.at[dst_v.at[NCH - 1]], add=True)
        cs_wait(NCH - 1, znr0, znc0, semr0, semc0)
        cos_compute(NCH - 1, znr0, znc0)

        plsc.subcore_barrier()
        pltpu.sync_copy(acc.at[pl.ds(off, RPT)],
                        out_hbm.at[c].at[pl.ds(off, RPT)])
        pltpu.sync_copy(wv, wmu_hbm.at[pl.ds(wid * EPW, EPW)])

    return gate_cos


_seg128 = _make_seg(128)
_seg64 = _make_seg(64)
_seg16 = _make_seg(16)
_gate_cos = _make_gate_cos()


def _leaky(v):
    return jnp.where(v >= 0, v, 0.2 * v)


def _dot(a, b):
    return jnp.dot(a, b, preferred_element_type=f32)


def _tc1(p_ref, x_ref, wr_ref, wt_ref, b_ref, ws_ref, bs_ref,
         h_ref, zstd_ref):
    agg = p_ref[0, :N] + p_ref[1, :N]
    h = _leaky(_dot(agg, wr_ref[...]) + _dot(x_ref[...], wt_ref[...])
               + b_ref[...][None, :])
    h_ref[...] = h
    zstd_ref[...] = jnp.exp(jnp.tanh(_dot(h, ws_ref[...])
                                     + bs_ref[...][None, :]))


def _tc2(p_ref, h_ref, wr_ref, wt_ref, b_ref, z_ref, zn_ref):
    agg = p_ref[0, :N] + p_ref[1, :N]
    z = jnp.tanh(_dot(agg, wr_ref[...]) + _dot(h_ref[...], wt_ref[...])
                 + b_ref[...][None, :])
    z_ref[...] = z
    na = jnp.maximum(jnp.sqrt(jnp.sum(z * z, axis=1, keepdims=True)), 1e-8)
    zn_ref[...] = z / na


def _tc3(p_ref, z_ref, wr_ref, wt_ref, b_ref, x1_ref):
    agg = p_ref[0, :N] + p_ref[1, :N]
    x1_ref[...] = _leaky(_dot(agg, wr_ref[...]) + _dot(z_ref[...], wt_ref[...])
                         + b_ref[...][None, :])


def _tc4(p_ref, x1_ref, wr_ref, wt_ref, b_ref, z_ref, wc1_ref, bc1_ref,
         wc2_ref, bc2_ref, lsp_ref, y_ref, wstd_ref):
    agg = p_ref[0, :N] + p_ref[1, :N]
    x1 = x1_ref[...]
    gate = (_dot(agg, wr_ref[...]) + _dot(x1, wt_ref[...])
            + b_ref[...][None, :])
    m = jnp.max(gate)
    eg = jnp.exp(gate - m)
    ssum = jnp.sum(eg)
    z = z_ref[...]
    pooled = (jnp.sum(eg * z, axis=0) / ssum).reshape(1, 32)
    y1 = _leaky(_dot(pooled, wc1_ref[...]) + bc1_ref[...][None, :])
    y2 = _dot(y1, wc2_ref[...]) + bc2_ref[...][None, :]
    y2 = y2 - jnp.max(y2, axis=1, keepdims=True)
    ey = jnp.exp(y2)
    y_ref[...] = ey / jnp.sum(ey, axis=1, keepdims=True)
    wstd_ref[...] = jnp.exp(lsp_ref[...])


def _sds(shape):
    return jax.ShapeDtypeStruct(shape, f32)


def kernel(x, edge_index, W1_rel, W1_root, b1, Wmu_rel, Wmu_root, bmu,
           Wstd, bstd, Wg1_rel, Wg1_root, bg1, Wg2_rel, Wg2_root, bg2,
           Wc1, bc1, Wc2, bc2, log_std_p):
    ei = edge_index.astype(jnp.int32)
    src = ei[1].reshape(NW, NCH, CH)
    dst = ei[0].reshape(NW, NCH, CH)
    z128 = jnp.zeros((NP, 128), f32)
    z64 = jnp.zeros((NP, 64), f32)
    z32 = jnp.zeros((NP, 32), f32)
    z16 = jnp.zeros((NP, 16), f32)

    p1 = _seg128(x, src, dst, z128)
    h, z_std = pl.pallas_call(
        _tc1, out_shape=(_sds((N, 64)), _sds((N, 32))))(
        p1, x, W1_rel, W1_root, b1, Wstd, bstd)
    p2 = _seg64(h, src, dst, z64)
    z, zn = pl.pallas_call(
        _tc2, out_shape=(_sds((N, 32)), _sds((N, ZNW))))(
        p2, h, Wmu_rel, Wmu_root, bmu)
    p3, w_mu = _gate_cos(z, zn, src, dst, z32)
    x1 = pl.pallas_call(
        _tc3, out_shape=_sds((N, 16)))(p3, z, Wg1_rel, Wg1_root, bg1)
    p4 = _seg16(x1, src, dst, z16)
    y, w_std = pl.pallas_call(
        _tc4, out_shape=(_sds((1, 2)), _sds((1,))))(
        p4, x1, Wg2_rel, Wg2_root, bg2, z, Wc1, bc1, Wc2, bc2, log_std_p)
    return (y, w_mu, w_std, z, z, z_std)
